# Initial kernel scaffold; baseline (speedup 1.0000x reference)
import jax
import jax.numpy as jnp
from jax.experimental import pallas as pl


def kernel(x):
    raise NotImplementedError("write your pallas kernel here")


# speedup vs baseline: 64.1221x; 64.1221x over previous
import functools

import jax
import jax.numpy as jnp
from jax import lax
from jax.experimental import pallas as pl
from jax.experimental.pallas import tpu as pltpu
from jax.experimental.pallas import tpu_sc as plsc

PERCENTILE = 0.999
N = 2 * 4096 * 2048
NW = 32
PER_W = N // NW
CHUNK = 8192
NCHUNKS = PER_W // CHUNK
NBINS = 65536
L = 16

_H_MAX = PERCENTILE * (N - 1)
_H_MIN = (1.0 - PERCENTILE) * (N - 1)
RANK_MAX = int(_H_MAX)
FRAC_MAX = _H_MAX - RANK_MAX
RANK_MIN = int(_H_MIN)
FRAC_MIN = _H_MIN - RANK_MIN

_SIGN = -2147483648


def _sc_histogram(x_i32):
  mesh = plsc.VectorSubcoreMesh(core_axis_name="c", subcore_axis_name="s")

  @functools.partial(
      pl.kernel,
      out_type=jax.ShapeDtypeStruct((NW, NBINS), jnp.int32),
      mesh=mesh,
      scratch_types=[
          pltpu.VMEM((NBINS,), jnp.int32),
          pltpu.VMEM((CHUNK,), jnp.int32),
          pltpu.VMEM((CHUNK,), jnp.int32),
          pltpu.SemaphoreType.DMA,
          pltpu.SemaphoreType.DMA,
      ],
      compiler_params=pltpu.CompilerParams(needs_layout_passes=False),
  )
  def hist_kernel(x_hbm, out_hbm, hist, buf0, buf1, sem0, sem1):
    nc = 2
    wid = lax.axis_index("s") * nc + lax.axis_index("c")
    base = wid * PER_W

    zeros = jnp.zeros((L,), jnp.int32)

    def zero_body(i, carry):
      hist[pl.ds(i * L, L)] = zeros
      return carry

    lax.fori_loop(0, NBINS // L, zero_body, 0, unroll=8)

    ones = jnp.full((L,), 1, jnp.int32)

    def bin_body(buf):
      def vec_body(j, carry):
        u = buf[pl.ds(j * L, L)]
        m = lax.shift_right_arithmetic(u, 31)
        key = lax.bitwise_xor(u, lax.bitwise_or(m, jnp.int32(_SIGN)))
        b = lax.shift_right_logical(key, 16)
        plsc.addupdate_scatter(hist, [b], ones)
        return carry

      lax.fori_loop(0, CHUNK // L, vec_body, 0, unroll=8)

    pltpu.async_copy(x_hbm.at[pl.ds(base, CHUNK)], buf0, sem0)
    pltpu.async_copy(x_hbm.at[pl.ds(base + CHUNK, CHUNK)], buf1, sem1)

    def round_body(r, carry):
      c0 = r * 2

      pltpu.make_async_copy(x_hbm.at[pl.ds(base, CHUNK)], buf0, sem0).wait()
      bin_body(buf0)

      @pl.when(c0 + 2 < NCHUNKS)
      def _():
        pltpu.async_copy(
            x_hbm.at[pl.ds(base + (c0 + 2) * CHUNK, CHUNK)], buf0, sem0)

      pltpu.make_async_copy(x_hbm.at[pl.ds(base, CHUNK)], buf1, sem1).wait()
      bin_body(buf1)

      @pl.when(c0 + 3 < NCHUNKS)
      def _():
        pltpu.async_copy(
            x_hbm.at[pl.ds(base + (c0 + 3) * CHUNK, CHUNK)], buf1, sem1)

      return carry

    lax.fori_loop(0, NCHUNKS // 2, round_body, 0)

    pltpu.sync_copy(hist, out_hbm.at[wid])

  return hist_kernel(x_i32)


def _cumsum_lanes(h):
  k = 1
  while k < h.shape[1]:
    shifted = jnp.concatenate(
        [jnp.zeros((h.shape[0], k), jnp.int32), h[:, :-k]], axis=1)
    h = h + shifted
    k *= 2
  return h


def _cumsum_rows(h):
  k = 1
  while k < h.shape[0]:
    shifted = jnp.concatenate(
        [jnp.zeros((k, h.shape[1]), jnp.int32), h[:-k, :]], axis=0)
    h = h + shifted
    k *= 2
  return h


def _decode_key(key):
  u = jnp.where(key < 0, key & jnp.int32(0x7FFFFFFF), ~key)
  return lax.bitcast_convert_type(u, jnp.float32)


def _tc_select(hist3):
  rows = NBINS // 128

  def select_kernel(h_ref, min_ref, max_ref):
    h = jnp.sum(h_ref[...], axis=0)
    cum_lane = _cumsum_lanes(h)
    row_tot = cum_lane[:, -1:]
    row_excl = _cumsum_rows(row_tot) - row_tot
    cinc = row_excl + cum_lane
    cexc = cinc - h

    r_idx = lax.broadcasted_iota(jnp.int32, (rows, 128), 0)
    c_idx = lax.broadcasted_iota(jnp.int32, (rows, 128), 1)
    bin_idx = r_idx * 128 + c_idx

    def quantile_at(rank, frac):
      rank = jnp.int32(rank)
      mask = (cexc <= rank) & (rank < cinc)
      b = jnp.max(jnp.where(mask, bin_idx, 0))
      cnt = jnp.max(jnp.where(mask, h, 0))
      ce = jnp.max(jnp.where(mask, cexc, 0))
      lo = _decode_key(b << 16)
      hi = _decode_key((b + 1) << 16)
      t = ((rank - ce).astype(jnp.float32) + jnp.float32(frac + 0.5)) / (
          cnt.astype(jnp.float32))
      t = jnp.clip(t, 0.0, 1.0)
      return lo + t * (hi - lo)

    min_ref[...] = jnp.broadcast_to(quantile_at(RANK_MIN, FRAC_MIN), (1, 1))
    max_ref[...] = jnp.broadcast_to(quantile_at(RANK_MAX, FRAC_MAX), (1, 1))

  return pl.pallas_call(
      select_kernel,
      out_shape=(
          jax.ShapeDtypeStruct((1, 1), jnp.float32),
          jax.ShapeDtypeStruct((1, 1), jnp.float32),
      ),
  )(hist3)


def kernel(x):
  x_i32 = lax.bitcast_convert_type(x, jnp.int32).reshape(N)
  hist = _sc_histogram(x_i32)
  minv, maxv = _tc_select(hist.reshape(NW, NBINS // 128, 128))
  return (x, minv.reshape(()), maxv.reshape(()))


# speedup vs baseline: 91.4972x; 1.4269x over previous
import functools

import jax
import jax.numpy as jnp
from jax import lax
from jax.experimental import pallas as pl
from jax.experimental.pallas import tpu as pltpu
from jax.experimental.pallas import tpu_sc as plsc

PERCENTILE = 0.999
N = 2 * 4096 * 2048
NW = 32
PER_W = N // NW
CHUNK = 8192
NCHUNKS = PER_W // CHUNK
NBINS = 65536
L = 16

_H_MAX = PERCENTILE * (N - 1)
_H_MIN = (1.0 - PERCENTILE) * (N - 1)
RANK_MAX = int(_H_MAX)
FRAC_MAX = _H_MAX - RANK_MAX
RANK_MIN = int(_H_MIN)
FRAC_MIN = _H_MIN - RANK_MIN

_SIGN = -2147483648


def _sc_histogram(x_i32):
  mesh = plsc.VectorSubcoreMesh(core_axis_name="c", subcore_axis_name="s")

  @functools.partial(
      pl.kernel,
      out_type=jax.ShapeDtypeStruct((NW, NBINS), jnp.int32),
      mesh=mesh,
      scratch_types=[
          pltpu.VMEM((NBINS,), jnp.int32),
          pltpu.VMEM((CHUNK,), jnp.int32),
          pltpu.VMEM((CHUNK,), jnp.int32),
          pltpu.SemaphoreType.DMA,
          pltpu.SemaphoreType.DMA,
      ],
      compiler_params=pltpu.CompilerParams(needs_layout_passes=False),
  )
  def hist_kernel(x_hbm, out_hbm, hist, buf0, buf1, sem0, sem1):
    nc = 2
    wid = lax.axis_index("s") * nc + lax.axis_index("c")
    base = wid * PER_W

    zeros = jnp.zeros((L,), jnp.int32)

    def zero_body(i, carry):
      hist[pl.ds(i * L, L)] = zeros
      return carry

    lax.fori_loop(0, NBINS // L, zero_body, 0, unroll=8)

    ones = jnp.full((L,), 1, jnp.int32)
    ilv = 16

    def bin_body(buf):
      def vec_body(j, carry):
        base_e = j * (L * ilv)
        us = [buf[pl.ds(base_e + k * L, L)] for k in range(ilv)]
        bs = [lax.shift_right_logical(u, 16) for u in us]
        for b in bs:
          plsc.addupdate_scatter(hist, [b], ones)
        return carry

      lax.fori_loop(0, CHUNK // (L * ilv), vec_body, 0, unroll=1)

    pltpu.async_copy(x_hbm.at[pl.ds(base, CHUNK)], buf0, sem0)
    pltpu.async_copy(x_hbm.at[pl.ds(base + CHUNK, CHUNK)], buf1, sem1)

    def round_body(r, carry):
      c0 = r * 2

      pltpu.make_async_copy(x_hbm.at[pl.ds(base, CHUNK)], buf0, sem0).wait()
      bin_body(buf0)

      @pl.when(c0 + 2 < NCHUNKS)
      def _():
        pltpu.async_copy(
            x_hbm.at[pl.ds(base + (c0 + 2) * CHUNK, CHUNK)], buf0, sem0)

      pltpu.make_async_copy(x_hbm.at[pl.ds(base, CHUNK)], buf1, sem1).wait()
      bin_body(buf1)

      @pl.when(c0 + 3 < NCHUNKS)
      def _():
        pltpu.async_copy(
            x_hbm.at[pl.ds(base + (c0 + 3) * CHUNK, CHUNK)], buf1, sem1)

      return carry

    lax.fori_loop(0, NCHUNKS // 2, round_body, 0)

    pltpu.sync_copy(hist, out_hbm.at[wid])

  return hist_kernel(x_i32)


def _cumsum_lanes(h):
  k = 1
  while k < h.shape[1]:
    shifted = jnp.concatenate(
        [jnp.zeros((h.shape[0], k), jnp.int32), h[:, :-k]], axis=1)
    h = h + shifted
    k *= 2
  return h


def _cumsum_rows(h):
  k = 1
  while k < h.shape[0]:
    shifted = jnp.concatenate(
        [jnp.zeros((k, h.shape[1]), jnp.int32), h[:-k, :]], axis=0)
    h = h + shifted
    k *= 2
  return h


def _decode_key(key):
  u = jnp.where(key < 0, key & jnp.int32(0x7FFFFFFF), ~key)
  return lax.bitcast_convert_type(u, jnp.float32)


def _tc_select(hist3):
  rows = NBINS // 128

  def select_kernel(h_ref, min_ref, max_ref):
    h = jnp.sum(h_ref[...], axis=0)
    cum_lane = _cumsum_lanes(h)
    row_tot = cum_lane[:, -1:]
    row_excl = _cumsum_rows(row_tot) - row_tot
    cinc = row_excl + cum_lane
    cexc = cinc - h

    r_idx = lax.broadcasted_iota(jnp.int32, (rows, 128), 0)
    c_idx = lax.broadcasted_iota(jnp.int32, (rows, 128), 1)
    bin_idx = r_idx * 128 + c_idx

    def quantile_at(rank, frac):
      rank = jnp.int32(rank)
      mask = (cexc <= rank) & (rank < cinc)
      b = jnp.max(jnp.where(mask, bin_idx, 0))
      cnt = jnp.max(jnp.where(mask, h, 0))
      ce = jnp.max(jnp.where(mask, cexc, 0))
      lo = _decode_key(b << 16)
      hi = _decode_key((b + 1) << 16)
      t = ((rank - ce).astype(jnp.float32) + jnp.float32(frac + 0.5)) / (
          cnt.astype(jnp.float32))
      t = jnp.clip(t, 0.0, 1.0)
      return lo + t * (hi - lo)

    min_ref[...] = jnp.broadcast_to(quantile_at(RANK_MIN, FRAC_MIN), (1, 1))
    max_ref[...] = jnp.broadcast_to(quantile_at(RANK_MAX, FRAC_MAX), (1, 1))

  return pl.pallas_call(
      select_kernel,
      out_shape=(
          jax.ShapeDtypeStruct((1, 1), jnp.float32),
          jax.ShapeDtypeStruct((1, 1), jnp.float32),
      ),
  )(hist3)


def _tc_copy(x):
  def copy_kernel(x_ref, o_ref):
    o_ref[...] = x_ref[...]

  return pl.pallas_call(
      copy_kernel,
      grid=(8,),
      in_specs=[pl.BlockSpec((2, 512, 2048), lambda i: (0, i, 0))],
      out_specs=pl.BlockSpec((2, 512, 2048), lambda i: (0, i, 0)),
      out_shape=jax.ShapeDtypeStruct(x.shape, x.dtype),
  )(x)


def kernel(x):
  x_i32 = lax.bitcast_convert_type(x, jnp.int32).reshape(N)
  hist_raw = _sc_histogram(x_i32)
  hist = jnp.concatenate(
      [jnp.flip(hist_raw[:, NBINS // 2:], axis=1), hist_raw[:, :NBINS // 2]],
      axis=1)
  minv, maxv = _tc_select(hist.reshape(NW, NBINS // 128, 128))
  return (_tc_copy(x), minv.reshape(()), maxv.reshape(()))


# speedup vs baseline: 180.8194x; 1.9762x over previous
import functools

import jax
import jax.numpy as jnp
from jax import lax
from jax.experimental import pallas as pl
from jax.experimental.pallas import tpu as pltpu
from jax.experimental.pallas import tpu_sc as plsc

PERCENTILE = 0.999
N = 2 * 4096 * 2048
ROWS, COLS = 8192, 2048
NW = 32
ROWS_W = ROWS // NW
CHUNK_R = 8
NCHUNKS = ROWS_W // CHUNK_R
NBINS = 65536
L = 16

_H_MAX = PERCENTILE * (N - 1)
_H_MIN = (1.0 - PERCENTILE) * (N - 1)
RANK_MAX = int(_H_MAX)
FRAC_MAX = _H_MAX - RANK_MAX
RANK_MIN = int(_H_MIN)
FRAC_MIN = _H_MIN - RANK_MIN


def _sc_histogram(x_i32):
  mesh = plsc.VectorSubcoreMesh(core_axis_name="c", subcore_axis_name="s")

  @functools.partial(
      pl.kernel,
      out_type=jax.ShapeDtypeStruct((NW, NBINS), jnp.int32),
      mesh=mesh,
      scratch_types=[
          pltpu.VMEM((NBINS,), jnp.int32),
          pltpu.VMEM((CHUNK_R, COLS), jnp.int32),
          pltpu.VMEM((CHUNK_R, COLS), jnp.int32),
          pltpu.SemaphoreType.DMA,
          pltpu.SemaphoreType.DMA,
      ],
      compiler_params=pltpu.CompilerParams(
          needs_layout_passes=False, use_tc_tiling_on_sc=True),
  )
  def hist_kernel(x_hbm, out_hbm, hist, buf0, buf1, sem0, sem1):
    nc = 2
    wid = lax.axis_index("s") * nc + lax.axis_index("c")
    row0 = wid * ROWS_W

    zeros = jnp.zeros((L,), jnp.int32)

    def zero_body(i, carry):
      hist[pl.ds(i * L, L)] = zeros
      return carry

    lax.fori_loop(0, NBINS // L, zero_body, 0, unroll=8)

    ones = jnp.full((L,), 1, jnp.int32)
    ilv = 16

    def bin_body(buf):
      for r in range(CHUNK_R):
        def vec_body(j, carry):
          base_e = j * (L * ilv)
          us = [buf[r, pl.ds(base_e + k * L, L)] for k in range(ilv)]
          bs = [lax.shift_right_logical(u, 16) for u in us]
          for b in bs:
            plsc.addupdate_scatter(hist, [b], ones)
          return carry

        lax.fori_loop(0, COLS // (L * ilv), vec_body, 0, unroll=1)

    pltpu.async_copy(x_hbm.at[pl.ds(row0, CHUNK_R)], buf0, sem0)
    pltpu.async_copy(x_hbm.at[pl.ds(row0 + CHUNK_R, CHUNK_R)], buf1, sem1)

    def round_body(rnd, carry):
      c0 = rnd * 2

      pltpu.make_async_copy(x_hbm.at[pl.ds(row0, CHUNK_R)], buf0, sem0).wait()
      bin_body(buf0)

      @pl.when(c0 + 2 < NCHUNKS)
      def _():
        pltpu.async_copy(
            x_hbm.at[pl.ds(row0 + (c0 + 2) * CHUNK_R, CHUNK_R)], buf0, sem0)

      pltpu.make_async_copy(x_hbm.at[pl.ds(row0, CHUNK_R)], buf1, sem1).wait()
      bin_body(buf1)

      @pl.when(c0 + 3 < NCHUNKS)
      def _():
        pltpu.async_copy(
            x_hbm.at[pl.ds(row0 + (c0 + 3) * CHUNK_R, CHUNK_R)], buf1, sem1)

      return carry

    lax.fori_loop(0, NCHUNKS // 2, round_body, 0)

    pltpu.sync_copy(hist, out_hbm.at[wid])

  return hist_kernel(x_i32)


def _bwd_cumsum_lanes(h):
  k = 1
  while k < h.shape[1]:
    shifted = jnp.concatenate(
        [h[:, k:], jnp.zeros((h.shape[0], k), jnp.int32)], axis=1)
    h = h + shifted
    k *= 2
  return h


def _bwd_cumsum_rows(h):
  k = 1
  while k < h.shape[0]:
    shifted = jnp.concatenate(
        [h[k:, :], jnp.zeros((k, h.shape[1]), jnp.int32)], axis=0)
    h = h + shifted
    k *= 2
  return h


def _decode_key(key):
  u = jnp.where(key < 0, key & jnp.int32(0x7FFFFFFF), ~key)
  return lax.bitcast_convert_type(u, jnp.float32)


def _tc_select(hist3):
  rows = NBINS // 128

  def select_kernel(h_ref, min_ref, max_ref):
    h = jnp.sum(h_ref[...], axis=0)
    bwd_lane = _bwd_cumsum_lanes(h)
    row_tot = bwd_lane[:, :1]
    row_suf_excl = _bwd_cumsum_rows(row_tot) - row_tot
    bwd_inc = row_suf_excl + bwd_lane

    r_idx = lax.broadcasted_iota(jnp.int32, (rows, 128), 0)
    c_idx = lax.broadcasted_iota(jnp.int32, (rows, 128), 1)
    raw = r_idx * 128 + c_idx
    isneg = raw >= NBINS // 2
    negtotal = bwd_inc[(NBINS // 2) // 128:(NBINS // 2) // 128 + 1, 0:1]
    cinc = jnp.where(isneg, bwd_inc, N - bwd_inc + h + negtotal)
    cexc = cinc - h
    jgrid = jnp.where(isneg, (NBINS - 1) - raw, raw + NBINS // 2)

    def quantile_at(rank, frac):
      rank = jnp.int32(rank)
      mask = (cexc <= rank) & (rank < cinc)
      b = jnp.max(jnp.where(mask, jgrid, 0))
      cnt = jnp.max(jnp.where(mask, h, 0))
      ce = jnp.max(jnp.where(mask, cexc, 0))
      lo = _decode_key(b << 16)
      hi = _decode_key((b + 1) << 16)
      t = ((rank - ce).astype(jnp.float32) + jnp.float32(frac + 0.5)) / (
          cnt.astype(jnp.float32))
      t = jnp.clip(t, 0.0, 1.0)
      return lo + t * (hi - lo)

    min_ref[...] = jnp.broadcast_to(quantile_at(RANK_MIN, FRAC_MIN), (1, 1))
    max_ref[...] = jnp.broadcast_to(quantile_at(RANK_MAX, FRAC_MAX), (1, 1))

  return pl.pallas_call(
      select_kernel,
      out_shape=(
          jax.ShapeDtypeStruct((1, 1), jnp.float32),
          jax.ShapeDtypeStruct((1, 1), jnp.float32),
      ),
  )(hist3)


def _tc_copy(x):
  def copy_kernel(x_ref, o_ref):
    o_ref[...] = x_ref[...]

  return pl.pallas_call(
      copy_kernel,
      grid=(8,),
      in_specs=[pl.BlockSpec((2, 512, 2048), lambda i: (0, i, 0))],
      out_specs=pl.BlockSpec((2, 512, 2048), lambda i: (0, i, 0)),
      out_shape=jax.ShapeDtypeStruct(x.shape, x.dtype),
  )(x)


def kernel(x):
  x_i32 = lax.bitcast_convert_type(x, jnp.int32).reshape(ROWS, COLS)
  hist_raw = _sc_histogram(x_i32)
  minv, maxv = _tc_select(hist_raw.reshape(NW, NBINS // 128, 128))
  return (_tc_copy(x), minv.reshape(()), maxv.reshape(()))


# speedup vs baseline: 212.8530x; 1.1772x over previous
import functools

import jax
import jax.numpy as jnp
from jax import lax
from jax.experimental import pallas as pl
from jax.experimental.pallas import tpu as pltpu
from jax.experimental.pallas import tpu_sc as plsc

PERCENTILE = 0.999
N = 2 * 4096 * 2048
ROWS, COLS = 8192, 2048
NW = 32
ROWS_W = ROWS // NW
CHUNK_R = 8
NCHUNKS = ROWS_W // CHUNK_R
NBINS = 65536
L = 16

_H_MAX = PERCENTILE * (N - 1)
_H_MIN = (1.0 - PERCENTILE) * (N - 1)
RANK_MAX = int(_H_MAX)
FRAC_MAX = _H_MAX - RANK_MAX
RANK_MIN = int(_H_MIN)
FRAC_MIN = _H_MIN - RANK_MIN


def _sc_histogram(x_i32):
  mesh = plsc.VectorSubcoreMesh(core_axis_name="c", subcore_axis_name="s")

  @functools.partial(
      pl.kernel,
      out_type=jax.ShapeDtypeStruct((NW, NBINS // 128, 128), jnp.int32),
      mesh=mesh,
      scratch_types=[
          pltpu.VMEM((NBINS // 128, 128), jnp.int32),
          pltpu.VMEM((CHUNK_R, COLS), jnp.int32),
          pltpu.VMEM((CHUNK_R, COLS), jnp.int32),
          pltpu.SemaphoreType.DMA,
          pltpu.SemaphoreType.DMA,
      ],
      compiler_params=pltpu.CompilerParams(
          needs_layout_passes=False, use_tc_tiling_on_sc=True),
  )
  def hist_kernel(x_hbm, out_hbm, hist, buf0, buf1, sem0, sem1):
    nc = 2
    wid = lax.axis_index("s") * nc + lax.axis_index("c")
    row0 = wid * ROWS_W

    zeros = jnp.zeros((L,), jnp.int32)

    def zero_body(i, carry):
      hist[i, pl.ds(0, L)] = zeros
      hist[i, pl.ds(L, L)] = zeros
      hist[i, pl.ds(2 * L, L)] = zeros
      hist[i, pl.ds(3 * L, L)] = zeros
      hist[i, pl.ds(4 * L, L)] = zeros
      hist[i, pl.ds(5 * L, L)] = zeros
      hist[i, pl.ds(6 * L, L)] = zeros
      hist[i, pl.ds(7 * L, L)] = zeros
      return carry

    lax.fori_loop(0, NBINS // 128, zero_body, 0, unroll=4)

    ones = jnp.full((L,), 1, jnp.int32)
    ilv = 16

    def bin_body(buf):
      for r in range(CHUNK_R):
        def vec_body(j, carry):
          base_e = j * (L * ilv)
          us = [buf[r, pl.ds(base_e + k * L, L)] for k in range(ilv)]
          bs = [lax.shift_right_logical(u, 16) for u in us]
          rcs = [(lax.shift_right_logical(b, 7),
                  lax.bitwise_and(b, jnp.int32(127))) for b in bs]
          for br, bc in rcs:
            plsc.addupdate_scatter(hist, [br, bc], ones)
          return carry

        lax.fori_loop(0, COLS // (L * ilv), vec_body, 0, unroll=1)

    pltpu.async_copy(x_hbm.at[pl.ds(row0, CHUNK_R)], buf0, sem0)
    pltpu.async_copy(x_hbm.at[pl.ds(row0 + CHUNK_R, CHUNK_R)], buf1, sem1)

    def round_body(rnd, carry):
      c0 = rnd * 2

      pltpu.make_async_copy(x_hbm.at[pl.ds(row0, CHUNK_R)], buf0, sem0).wait()
      bin_body(buf0)

      @pl.when(c0 + 2 < NCHUNKS)
      def _():
        pltpu.async_copy(
            x_hbm.at[pl.ds(row0 + (c0 + 2) * CHUNK_R, CHUNK_R)], buf0, sem0)

      pltpu.make_async_copy(x_hbm.at[pl.ds(row0, CHUNK_R)], buf1, sem1).wait()
      bin_body(buf1)

      @pl.when(c0 + 3 < NCHUNKS)
      def _():
        pltpu.async_copy(
            x_hbm.at[pl.ds(row0 + (c0 + 3) * CHUNK_R, CHUNK_R)], buf1, sem1)

      return carry

    lax.fori_loop(0, NCHUNKS // 2, round_body, 0)

    pltpu.sync_copy(hist, out_hbm.at[wid])

  return hist_kernel(x_i32)


def _bwd_cumsum_lanes(h):
  k = 1
  while k < h.shape[1]:
    shifted = jnp.concatenate(
        [h[:, k:], jnp.zeros((h.shape[0], k), jnp.int32)], axis=1)
    h = h + shifted
    k *= 2
  return h


def _bwd_cumsum_rows(h):
  k = 1
  while k < h.shape[0]:
    shifted = jnp.concatenate(
        [h[k:, :], jnp.zeros((k, h.shape[1]), jnp.int32)], axis=0)
    h = h + shifted
    k *= 2
  return h


def _decode_key(key):
  u = jnp.where(key < 0, key & jnp.int32(0x7FFFFFFF), ~key)
  return lax.bitcast_convert_type(u, jnp.float32)


def _tc_select(hist3):
  rows = NBINS // 128

  def select_kernel(h_ref, min_ref, max_ref):
    h = jnp.sum(h_ref[...], axis=0)
    bwd_lane = _bwd_cumsum_lanes(h)
    row_tot = bwd_lane[:, :1]
    row_suf_excl = _bwd_cumsum_rows(row_tot) - row_tot
    bwd_inc = row_suf_excl + bwd_lane

    r_idx = lax.broadcasted_iota(jnp.int32, (rows, 128), 0)
    c_idx = lax.broadcasted_iota(jnp.int32, (rows, 128), 1)
    raw = r_idx * 128 + c_idx
    isneg = raw >= NBINS // 2
    negtotal = bwd_inc[(NBINS // 2) // 128:(NBINS // 2) // 128 + 1, 0:1]
    cinc = jnp.where(isneg, bwd_inc, N - bwd_inc + h + negtotal)
    cexc = cinc - h
    jgrid = jnp.where(isneg, (NBINS - 1) - raw, raw + NBINS // 2)

    def quantile_at(rank, frac):
      rank = jnp.int32(rank)
      mask = (cexc <= rank) & (rank < cinc)
      b = jnp.max(jnp.where(mask, jgrid, 0))
      cnt = jnp.max(jnp.where(mask, h, 0))
      ce = jnp.max(jnp.where(mask, cexc, 0))
      lo = _decode_key(b << 16)
      hi = _decode_key((b + 1) << 16)
      t = ((rank - ce).astype(jnp.float32) + jnp.float32(frac + 0.5)) / (
          cnt.astype(jnp.float32))
      t = jnp.clip(t, 0.0, 1.0)
      return lo + t * (hi - lo)

    min_ref[...] = jnp.broadcast_to(quantile_at(RANK_MIN, FRAC_MIN), (1, 1))
    max_ref[...] = jnp.broadcast_to(quantile_at(RANK_MAX, FRAC_MAX), (1, 1))

  return pl.pallas_call(
      select_kernel,
      out_shape=(
          jax.ShapeDtypeStruct((1, 1), jnp.float32),
          jax.ShapeDtypeStruct((1, 1), jnp.float32),
      ),
  )(hist3)


def _tc_copy(x):
  def copy_kernel(x_ref, o_ref):
    o_ref[...] = x_ref[...]

  return pl.pallas_call(
      copy_kernel,
      grid=(8,),
      in_specs=[pl.BlockSpec((2, 512, 2048), lambda i: (0, i, 0))],
      out_specs=pl.BlockSpec((2, 512, 2048), lambda i: (0, i, 0)),
      out_shape=jax.ShapeDtypeStruct(x.shape, x.dtype),
  )(x)


def kernel(x):
  xc = _tc_copy(x)
  x_i32 = lax.bitcast_convert_type(x, jnp.int32).reshape(ROWS, COLS)
  hist_raw = _sc_histogram(x_i32)
  minv, maxv = _tc_select(hist_raw)
  return (xc, minv.reshape(()), maxv.reshape(()))
